# baseline (speedup 1.0000x reference)
import functools

import jax
import jax.numpy as jnp
from jax import lax
from jax.experimental import pallas as pl
from jax.experimental.pallas import tpu as pltpu
from jax.experimental.pallas import tpu_sc as plsc

VOCAB = 1_000_000
DIM = 64

NC = 2
NS = 16
NW = NC * NS

CH = 128


def _emb_body(seq, n_chunks, w2, xT, outT, idx_v, idx2_v, cb_v, rows_v, tr_v,
              gsem):
    wid = lax.axis_index("s") * NC + lax.axis_index("c")
    tpw = (seq * n_chunks) // NW
    t_base = wid * tpw
    iota = lax.iota(jnp.int32, 16)

    @pl.loop(0, tpw)
    def _task(t):
        tt = t_base + t
        s = tt // n_chunks
        b0 = (tt % n_chunks) * CH
        pltpu.sync_copy(xT.at[s, pl.ds(b0, CH)], idx_v)
        for g in range(CH // 16):
            vi = idx_v[pl.ds(g * 16, 16)]
            idx2_v[pl.ds(g * 16, 16)] = lax.shift_right_logical(vi, 1)
            cb_v[pl.ds(g * 16, 16)] = lax.shift_left(
                lax.bitwise_and(vi, 1), 6)
        pltpu.async_copy(w2.at[idx2_v], rows_v, gsem).wait()

        @pl.loop(0, DIM)
        def _d(d):
            dv = jnp.broadcast_to(d, (16,))
            for g in range(CH // 16):
                rid = g * 16 + iota
                col = cb_v[pl.ds(g * 16, 16)] + dv
                tr_v[d, pl.ds(g * 16, 16)] = plsc.load_gather(
                    rows_v, [rid, col])

        pltpu.sync_copy(tr_v, outT.at[s, :, pl.ds(b0, CH)])


@functools.partial(jax.jit, static_argnames=("seq", "n_chunks"))
def _emb(w2, xT, seq, n_chunks):
    mesh = plsc.VectorSubcoreMesh(
        core_axis_name="c", subcore_axis_name="s", num_cores=NC, num_subcores=NS
    )
    batch = n_chunks * CH
    return pl.kernel(
        functools.partial(_emb_body, seq, n_chunks),
        out_type=jax.ShapeDtypeStruct((seq, DIM, batch), jnp.float32),
        mesh=mesh,
        scratch_types=[
            pltpu.VMEM((CH,), jnp.int32),
            pltpu.VMEM((CH,), jnp.int32),
            pltpu.VMEM((CH,), jnp.int32),
            pltpu.VMEM((CH, 128), jnp.float32),
            pltpu.VMEM((DIM, CH), jnp.float32),
            pltpu.SemaphoreType.DMA,
        ],
        compiler_params=pltpu.CompilerParams(
            use_tc_tiling_on_sc=True, needs_layout_passes=False),
    )(w2, xT)


def kernel(x, weight):
    b, s = x.shape
    assert b % CH == 0 and (s * (b // CH)) % NW == 0
    w2 = jnp.reshape(weight, (VOCAB // 2, 2 * DIM))
    xT = x.T.astype(jnp.int32)
    outT = _emb(w2, xT, s, b // CH)
    return outT.transpose(2, 0, 1)


# speedup vs baseline: 1.3028x; 1.3028x over previous
import functools

import jax
import jax.numpy as jnp
from jax import lax
from jax.experimental import pallas as pl
from jax.experimental.pallas import tpu as pltpu
from jax.experimental.pallas import tpu_sc as plsc

VOCAB = 1_000_000
DIM = 64

NC = 2
NS = 16
NW = NC * NS

CH = 128


def _emb_body(seq, n_chunks, w2, xT, outT, idx_v, idx2_v, rows_v, tr_v,
              gsem):
    wid = lax.axis_index("s") * NC + lax.axis_index("c")
    tpw = (seq * n_chunks) // NW
    t_base = wid * tpw
    iota = lax.iota(jnp.int32, 16)

    @pl.loop(0, tpw)
    def _task(t):
        tt = t_base + t
        s = tt // n_chunks
        b0 = (tt % n_chunks) * CH
        pltpu.sync_copy(xT.at[s, pl.ds(b0, CH)], idx_v)
        cbs = []
        for g in range(CH // 16):
            vi = idx_v[pl.ds(g * 16, 16)]
            idx2_v[pl.ds(g * 16, 16)] = lax.shift_right_logical(vi, 1)
            cbs.append(lax.shift_left(lax.bitwise_and(vi, 1), 6))
        pltpu.async_copy(w2.at[idx2_v], rows_v, gsem).wait()

        for g in range(CH // 16):
            rid = g * 16 + iota
            for d in range(DIM):
                tr_v[d, pl.ds(g * 16, 16)] = plsc.load_gather(
                    rows_v, [rid, cbs[g] + d])

        pltpu.sync_copy(tr_v, outT.at[s, :, pl.ds(b0, CH)])


@functools.partial(jax.jit, static_argnames=("seq", "n_chunks"))
def _emb(w2, xT, seq, n_chunks):
    mesh = plsc.VectorSubcoreMesh(
        core_axis_name="c", subcore_axis_name="s", num_cores=NC, num_subcores=NS
    )
    batch = n_chunks * CH
    return pl.kernel(
        functools.partial(_emb_body, seq, n_chunks),
        out_type=jax.ShapeDtypeStruct((seq, DIM, batch), jnp.float32),
        mesh=mesh,
        scratch_types=[
            pltpu.VMEM((CH,), jnp.int32),
            pltpu.VMEM((CH,), jnp.int32),
            pltpu.VMEM((CH, 128), jnp.float32),
            pltpu.VMEM((DIM, CH), jnp.float32),
            pltpu.SemaphoreType.DMA,
        ],
        compiler_params=pltpu.CompilerParams(
            use_tc_tiling_on_sc=True, needs_layout_passes=False),
    )(w2, xT)


def kernel(x, weight):
    b, s = x.shape
    assert b % CH == 0 and (s * (b // CH)) % NW == 0
    w2 = jnp.reshape(weight, (VOCAB // 2, 2 * DIM))
    xT = x.T.astype(jnp.int32)
    outT = _emb(w2, xT, s, b // CH)
    return outT.transpose(2, 0, 1)


# speedup vs baseline: 1.8124x; 1.3912x over previous
import functools

import jax
import jax.numpy as jnp
from jax import lax
from jax.experimental import pallas as pl
from jax.experimental.pallas import tpu as pltpu
from jax.experimental.pallas import tpu_sc as plsc

VOCAB = 1_000_000
DIM = 64

NC = 2
NS = 16
NW = NC * NS

CH = 128


def _emb_body(seq, n_chunks, w2, xT, outT, ia_all,
              idx2_0, idx2_1, rows0, rows1, tr0, tr1,
              gsem0, gsem1, osem0, osem1):
    wid = lax.axis_index("s") * NC + lax.axis_index("c")
    cpw = n_chunks // NW
    c_base = wid * cpw
    srows = (seq + 7) // 8 * 8
    iota = lax.iota(jnp.int32, 16)

    for j in range(cpw):
        pltpu.sync_copy(
            xT.at[:, pl.ds((c_base + j) * CH, CH)],
            ia_all.at[pl.ds(j * srows, seq), :],
        )

    def prep(row, idx2b, rowsb, gsemb):
        cbs = []
        for g in range(CH // 16):
            vi = ia_all[row, pl.ds(g * 16, 16)]
            idx2b[pl.ds(g * 16, 16)] = lax.shift_right_logical(vi, 1)
            cbs.append(lax.shift_left(lax.bitwise_and(vi, 1), 6))
        return cbs, pltpu.async_copy(w2.at[idx2b], rowsb, gsemb)

    def transpose(cbs, rowsb, trb):
        for g in range(CH // 16):
            rid = g * 16 + iota
            for d0 in range(0, DIM, 4):
                vs = [
                    plsc.load_gather(rowsb, [rid, cbs[g] + (d0 + k)])
                    for k in range(4)
                ]
                for k in range(4):
                    trb[d0 + k, pl.ds(g * 16, 16)] = vs[k]

    def fire_out(s, cglob, trb, osemb):
        return pltpu.async_copy(
            trb, outT.at[s, :, pl.ds(cglob * CH, CH)], osemb)

    def drain_out(trb, osemb):
        pltpu.make_async_copy(
            trb, outT.at[0, :, pl.ds(0, CH)], osemb).wait()

    def task_coords(t):
        j = t // seq
        s = t - j * seq
        return j, s

    @pl.loop(0, (cpw * seq) // 2)
    def _pair(i):
        t0 = 2 * i
        t1 = t0 + 1
        j0, s0 = task_coords(t0)
        j1, s1 = task_coords(t1)
        cbs0, d0 = prep(j0 * srows + s0, idx2_0, rows0, gsem0)
        cbs1, d1 = prep(j1 * srows + s1, idx2_1, rows1, gsem1)
        d0.wait()

        @pl.when(i > 0)
        def _():
            drain_out(tr0, osem0)

        transpose(cbs0, rows0, tr0)
        fire_out(s0, c_base + j0, tr0, osem0)
        d1.wait()

        @pl.when(i > 0)
        def _():
            drain_out(tr1, osem1)

        transpose(cbs1, rows1, tr1)
        fire_out(s1, c_base + j1, tr1, osem1)

    drain_out(tr0, osem0)
    drain_out(tr1, osem1)


@functools.partial(jax.jit, static_argnames=("seq", "n_chunks"))
def _emb(w2, xT, seq, n_chunks):
    mesh = plsc.VectorSubcoreMesh(
        core_axis_name="c", subcore_axis_name="s", num_cores=NC, num_subcores=NS
    )
    batch = n_chunks * CH
    srows = (seq + 7) // 8 * 8
    cpw = n_chunks // NW
    return pl.kernel(
        functools.partial(_emb_body, seq, n_chunks),
        out_type=jax.ShapeDtypeStruct((seq, DIM, batch), jnp.float32),
        mesh=mesh,
        scratch_types=[
            pltpu.VMEM((cpw * srows, CH), jnp.int32),
            pltpu.VMEM((CH,), jnp.int32),
            pltpu.VMEM((CH,), jnp.int32),
            pltpu.VMEM((CH, 128), jnp.float32),
            pltpu.VMEM((CH, 128), jnp.float32),
            pltpu.VMEM((DIM, CH), jnp.float32),
            pltpu.VMEM((DIM, CH), jnp.float32),
            pltpu.SemaphoreType.DMA,
            pltpu.SemaphoreType.DMA,
            pltpu.SemaphoreType.DMA,
            pltpu.SemaphoreType.DMA,
        ],
        compiler_params=pltpu.CompilerParams(
            use_tc_tiling_on_sc=True, needs_layout_passes=False),
    )(w2, xT)


def kernel(x, weight):
    b, s = x.shape
    assert b % CH == 0 and (b // CH) % NW == 0 and s % 2 == 0
    w2 = jnp.reshape(weight, (VOCAB // 2, 2 * DIM))
    xT = x.T.astype(jnp.int32)
    outT = _emb(w2, xT, s, b // CH)
    return outT.transpose(2, 0, 1)


# speedup vs baseline: 1.8561x; 1.0241x over previous
import functools

import jax
import jax.numpy as jnp
from jax import lax
from jax.experimental import pallas as pl
from jax.experimental.pallas import tpu as pltpu
from jax.experimental.pallas import tpu_sc as plsc

VOCAB = 1_000_000
DIM = 64

NC = 2
NS = 16
NW = NC * NS

CH = 128


def _emb_body(seq, n_chunks, w2, xT, outT, ia_all,
              idx2_0, idx2_1, idx2_2, idx2_3,
              rows0, rows1, rows2, rows3, tr0, tr1,
              gsem0, gsem1, gsem2, gsem3, osem0, osem1):
    wid = lax.axis_index("s") * NC + lax.axis_index("c")
    cpw = n_chunks // NW
    c_base = wid * cpw
    srows = (seq + 7) // 8 * 8
    iota = lax.iota(jnp.int32, 16)

    for j in range(cpw):
        pltpu.sync_copy(
            xT.at[:, pl.ds((c_base + j) * CH, CH)],
            ia_all.at[pl.ds(j * srows, seq), :],
        )

    def prep(row, idx2b, rowsb, gsemb):
        cbs = []
        for g in range(CH // 16):
            vi = ia_all[row, pl.ds(g * 16, 16)]
            idx2b[pl.ds(g * 16, 16)] = lax.shift_right_logical(vi, 1)
            cbs.append(lax.shift_left(lax.bitwise_and(vi, 1), 6))
        return cbs, pltpu.async_copy(w2.at[idx2b], rowsb, gsemb)

    def transpose(cbs, rowsb, trb):
        for g in range(CH // 16):
            rid = g * 16 + iota
            for d0 in range(0, DIM, 4):
                vs = [
                    plsc.load_gather(rowsb, [rid, cbs[g] + (d0 + k)])
                    for k in range(4)
                ]
                for k in range(4):
                    trb[d0 + k, pl.ds(g * 16, 16)] = vs[k]

    def fire_out(s, cglob, trb, osemb):
        return pltpu.async_copy(
            trb, outT.at[s, :, pl.ds(cglob * CH, CH)], osemb)

    def drain_out(trb, osemb):
        pltpu.make_async_copy(
            trb, outT.at[0, :, pl.ds(0, CH)], osemb).wait()

    def task_coords(t):
        j = t // seq
        s = t - j * seq
        return j, s

    idx2s = [idx2_0, idx2_1, idx2_2, idx2_3]
    rowss = [rows0, rows1, rows2, rows3]
    trs = [tr0, tr1]
    gsems = [gsem0, gsem1, gsem2, gsem3]
    osems = [osem0, osem1]

    @pl.loop(0, (cpw * seq) // 4)
    def _quad(i):
        staged = []
        for k in range(4):
            t = 4 * i + k
            j, s = task_coords(t)
            cbs, d = prep(j * srows + s, idx2s[k], rowss[k], gsems[k])
            staged.append((s, c_base + j, cbs, d))
        for k in range(4):
            s, cglob, cbs, d = staged[k]
            d.wait()
            if k < 2:
                @pl.when(i > 0)
                def _():
                    drain_out(trs[k % 2], osems[k % 2])
            else:
                drain_out(trs[k % 2], osems[k % 2])
            transpose(cbs, rowss[k], trs[k % 2])
            fire_out(s, cglob, trs[k % 2], osems[k % 2])

    drain_out(tr0, osem0)
    drain_out(tr1, osem1)


@functools.partial(jax.jit, static_argnames=("seq", "n_chunks"))
def _emb(w2, xT, seq, n_chunks):
    mesh = plsc.VectorSubcoreMesh(
        core_axis_name="c", subcore_axis_name="s", num_cores=NC, num_subcores=NS
    )
    batch = n_chunks * CH
    srows = (seq + 7) // 8 * 8
    cpw = n_chunks // NW
    return pl.kernel(
        functools.partial(_emb_body, seq, n_chunks),
        out_type=jax.ShapeDtypeStruct((seq, DIM, batch), jnp.float32),
        mesh=mesh,
        scratch_types=[
            pltpu.VMEM((cpw * srows, CH), jnp.int32),
            pltpu.VMEM((CH,), jnp.int32),
            pltpu.VMEM((CH,), jnp.int32),
            pltpu.VMEM((CH,), jnp.int32),
            pltpu.VMEM((CH,), jnp.int32),
            pltpu.VMEM((CH, 128), jnp.float32),
            pltpu.VMEM((CH, 128), jnp.float32),
            pltpu.VMEM((CH, 128), jnp.float32),
            pltpu.VMEM((CH, 128), jnp.float32),
            pltpu.VMEM((DIM, CH), jnp.float32),
            pltpu.VMEM((DIM, CH), jnp.float32),
            pltpu.SemaphoreType.DMA,
            pltpu.SemaphoreType.DMA,
            pltpu.SemaphoreType.DMA,
            pltpu.SemaphoreType.DMA,
            pltpu.SemaphoreType.DMA,
            pltpu.SemaphoreType.DMA,
        ],
        compiler_params=pltpu.CompilerParams(
            use_tc_tiling_on_sc=True, needs_layout_passes=False),
    )(w2, xT)


def kernel(x, weight):
    b, s = x.shape
    assert b % CH == 0 and (b // CH) % NW == 0 and s % 2 == 0
    w2 = jnp.reshape(weight, (VOCAB // 2, 2 * DIM))
    xT = x.T.astype(jnp.int32)
    outT = _emb(w2, xT, s, b // CH)
    return outT.transpose(2, 0, 1)


# speedup vs baseline: 2.2804x; 1.2286x over previous
import functools

import jax
import jax.numpy as jnp
from jax import lax
from jax.experimental import pallas as pl
from jax.experimental.pallas import tpu as pltpu
from jax.experimental.pallas import tpu_sc as plsc

VOCAB = 1_000_000
DIM = 64

NC = 2
NS = 16
NW = NC * NS

CH = 128


def _emb_body(seq, n_chunks, w2, xT, outT, ia_all,
              idx2_0, idx2_1, cb_0, cb_1,
              rows0, rows1, tr0, tr1,
              gsem0, gsem1, osem0, osem1):
    wid = lax.axis_index("s") * NC + lax.axis_index("c")
    cpw = n_chunks // NW
    c_base = wid * cpw
    srows = (seq + 7) // 8 * 8
    iota = lax.iota(jnp.int32, 16)

    for j in range(cpw):
        pltpu.sync_copy(
            xT.at[:, pl.ds((c_base + j) * CH, CH)],
            ia_all.at[pl.ds(j * srows, seq), :],
        )

    def prep(row, idx2b, cbb, rowsb, gsemb):
        for g in range(CH // 16):
            vi = ia_all[row, pl.ds(g * 16, 16)]
            idx2b[pl.ds(g * 16, 16)] = lax.shift_right_logical(vi, 1)
            cbb[pl.ds(g * 16, 16)] = lax.shift_left(
                lax.bitwise_and(vi, 1), 6)
        return pltpu.async_copy(w2.at[idx2b], rowsb, gsemb)

    def transpose(cbb, rowsb, trb):
        @pl.loop(0, CH // 16)
        def _g(g):
            g16 = g * 16
            rid = g16 + iota
            cb = cbb[pl.ds(g16, 16)]
            for d0 in range(0, DIM, 16):
                cbd = cb + d0
                for k in range(16):
                    perm = lax.bitwise_and(iota + k, 15)
                    vals = plsc.load_gather(rowsb, [rid, cbd + perm])
                    plsc.store_scatter(trb, [perm + d0, rid], vals)

    def fire_out(s, cglob, trb, osemb):
        return pltpu.async_copy(
            trb, outT.at[s, :, pl.ds(cglob * CH, CH)], osemb)

    def drain_out(trb, osemb):
        pltpu.make_async_copy(
            trb, outT.at[0, :, pl.ds(0, CH)], osemb).wait()

    def task_coords(t):
        j = t // seq
        s = t - j * seq
        return j, s

    idx2s = [idx2_0, idx2_1]
    cbbs = [cb_0, cb_1]
    rowss = [rows0, rows1]
    trs = [tr0, tr1]
    gsems = [gsem0, gsem1]
    osems = [osem0, osem1]

    @pl.loop(0, (cpw * seq) // 2)
    def _pair(i):
        staged = []
        for k in range(2):
            t = 2 * i + k
            j, s = task_coords(t)
            d = prep(j * srows + s, idx2s[k], cbbs[k], rowss[k], gsems[k])
            staged.append((s, c_base + j, d))
        for k in range(2):
            s, cglob, d = staged[k]
            d.wait()

            @pl.when(i > 0)
            def _():
                drain_out(trs[k], osems[k])

            transpose(cbbs[k], rowss[k], trs[k])
            fire_out(s, cglob, trs[k], osems[k])

    drain_out(tr0, osem0)
    drain_out(tr1, osem1)


@functools.partial(jax.jit, static_argnames=("seq", "n_chunks"))
def _emb(w2, xT, seq, n_chunks):
    mesh = plsc.VectorSubcoreMesh(
        core_axis_name="c", subcore_axis_name="s", num_cores=NC, num_subcores=NS
    )
    batch = n_chunks * CH
    srows = (seq + 7) // 8 * 8
    cpw = n_chunks // NW
    return pl.kernel(
        functools.partial(_emb_body, seq, n_chunks),
        out_type=jax.ShapeDtypeStruct((seq, DIM, batch), jnp.float32),
        mesh=mesh,
        scratch_types=[
            pltpu.VMEM((cpw * srows, CH), jnp.int32),
            pltpu.VMEM((CH,), jnp.int32),
            pltpu.VMEM((CH,), jnp.int32),
            pltpu.VMEM((CH,), jnp.int32),
            pltpu.VMEM((CH,), jnp.int32),
            pltpu.VMEM((CH, 128), jnp.float32),
            pltpu.VMEM((CH, 128), jnp.float32),
            pltpu.VMEM((DIM, CH), jnp.float32),
            pltpu.VMEM((DIM, CH), jnp.float32),
            pltpu.SemaphoreType.DMA,
            pltpu.SemaphoreType.DMA,
            pltpu.SemaphoreType.DMA,
            pltpu.SemaphoreType.DMA,
        ],
        compiler_params=pltpu.CompilerParams(
            use_tc_tiling_on_sc=True, needs_layout_passes=False),
    )(w2, xT)


def kernel(x, weight):
    b, s = x.shape
    assert b % CH == 0 and (b // CH) % NW == 0 and s % 2 == 0
    w2 = jnp.reshape(weight, (VOCAB // 2, 2 * DIM))
    xT = x.T.astype(jnp.int32)
    outT = _emb(w2, xT, s, b // CH)
    return outT.transpose(2, 0, 1)


# speedup vs baseline: 2.3187x; 1.0168x over previous
import functools

import jax
import jax.numpy as jnp
from jax import lax
from jax.experimental import pallas as pl
from jax.experimental.pallas import tpu as pltpu
from jax.experimental.pallas import tpu_sc as plsc

VOCAB = 1_000_000
DIM = 64

NC = 2
NS = 16
NW = NC * NS

CH = 128


def _emb_body(seq, n_chunks, w2, xT, outT, ia_all,
              idx2_0, idx2_1, idx2_2, idx2_3,
              cb_0, cb_1, cb_2, cb_3,
              rows0, rows1, rows2, rows3, tr0, tr1,
              gsem0, gsem1, gsem2, gsem3, osem0, osem1):
    wid = lax.axis_index("s") * NC + lax.axis_index("c")
    cpw = n_chunks // NW
    c_base = wid * cpw
    srows = (seq + 7) // 8 * 8
    iota = lax.iota(jnp.int32, 16)

    for j in range(cpw):
        pltpu.sync_copy(
            xT.at[:, pl.ds((c_base + j) * CH, CH)],
            ia_all.at[pl.ds(j * srows, seq), :],
        )

    def prep(row, idx2b, cbb, rowsb, gsemb):
        for g in range(CH // 16):
            vi = ia_all[row, pl.ds(g * 16, 16)]
            idx2b[pl.ds(g * 16, 16)] = lax.shift_right_logical(vi, 1)
            cbb[pl.ds(g * 16, 16)] = lax.shift_left(
                lax.bitwise_and(vi, 1), 6)
        return pltpu.async_copy(w2.at[idx2b], rowsb, gsemb)

    def transpose(cbb, rowsb, trb):
        @pl.loop(0, CH // 16)
        def _g(g):
            g16 = g * 16
            rid = g16 + iota
            cb = cbb[pl.ds(g16, 16)]
            for d0 in range(0, DIM, 16):
                cbd = cb + d0
                for k in range(16):
                    perm = lax.bitwise_and(iota + k, 15)
                    vals = plsc.load_gather(rowsb, [rid, cbd + perm])
                    plsc.store_scatter(trb, [perm + d0, rid], vals)

    def fire_out(s, cglob, trb, osemb):
        return pltpu.async_copy(
            trb, outT.at[s, :, pl.ds(cglob * CH, CH)], osemb)

    def drain_out(trb, osemb):
        pltpu.make_async_copy(
            trb, outT.at[0, :, pl.ds(0, CH)], osemb).wait()

    def task_coords(t):
        j = t // seq
        s = t - j * seq
        return j, s

    idx2s = [idx2_0, idx2_1, idx2_2, idx2_3]
    cbbs = [cb_0, cb_1, cb_2, cb_3]
    rowss = [rows0, rows1, rows2, rows3]
    trs = [tr0, tr1]
    gsems = [gsem0, gsem1, gsem2, gsem3]
    osems = [osem0, osem1]

    @pl.loop(0, (cpw * seq) // 4)
    def _quad(i):
        staged = []
        for k in range(4):
            t = 4 * i + k
            j, s = task_coords(t)
            d = prep(j * srows + s, idx2s[k], cbbs[k], rowss[k], gsems[k])
            staged.append((s, c_base + j, d))
        for k in range(4):
            s, cglob, d = staged[k]
            d.wait()
            if k < 2:
                @pl.when(i > 0)
                def _():
                    drain_out(trs[k % 2], osems[k % 2])
            else:
                drain_out(trs[k % 2], osems[k % 2])
            transpose(cbbs[k], rowss[k], trs[k % 2])
            fire_out(s, cglob, trs[k % 2], osems[k % 2])

    drain_out(tr0, osem0)
    drain_out(tr1, osem1)


@functools.partial(jax.jit, static_argnames=("seq", "n_chunks"))
def _emb(w2, xT, seq, n_chunks):
    mesh = plsc.VectorSubcoreMesh(
        core_axis_name="c", subcore_axis_name="s", num_cores=NC, num_subcores=NS
    )
    batch = n_chunks * CH
    srows = (seq + 7) // 8 * 8
    cpw = n_chunks // NW
    return pl.kernel(
        functools.partial(_emb_body, seq, n_chunks),
        out_type=jax.ShapeDtypeStruct((seq, DIM, batch), jnp.float32),
        mesh=mesh,
        scratch_types=[
            pltpu.VMEM((cpw * srows, CH), jnp.int32),
            pltpu.VMEM((CH,), jnp.int32),
            pltpu.VMEM((CH,), jnp.int32),
            pltpu.VMEM((CH,), jnp.int32),
            pltpu.VMEM((CH,), jnp.int32),
            pltpu.VMEM((CH,), jnp.int32),
            pltpu.VMEM((CH,), jnp.int32),
            pltpu.VMEM((CH,), jnp.int32),
            pltpu.VMEM((CH,), jnp.int32),
            pltpu.VMEM((CH, 128), jnp.float32),
            pltpu.VMEM((CH, 128), jnp.float32),
            pltpu.VMEM((CH, 128), jnp.float32),
            pltpu.VMEM((CH, 128), jnp.float32),
            pltpu.VMEM((DIM, CH), jnp.float32),
            pltpu.VMEM((DIM, CH), jnp.float32),
            pltpu.SemaphoreType.DMA,
            pltpu.SemaphoreType.DMA,
            pltpu.SemaphoreType.DMA,
            pltpu.SemaphoreType.DMA,
            pltpu.SemaphoreType.DMA,
            pltpu.SemaphoreType.DMA,
        ],
        compiler_params=pltpu.CompilerParams(
            use_tc_tiling_on_sc=True, needs_layout_passes=False),
    )(w2, xT)


def kernel(x, weight):
    b, s = x.shape
    assert b % CH == 0 and (b // CH) % NW == 0 and s % 2 == 0
    w2 = jnp.reshape(weight, (VOCAB // 2, 2 * DIM))
    xT = x.T.astype(jnp.int32)
    outT = _emb(w2, xT, s, b // CH)
    return outT.transpose(2, 0, 1)


# speedup vs baseline: 2.4392x; 1.0520x over previous
import functools

import jax
import jax.numpy as jnp
from jax import lax
from jax.experimental import pallas as pl
from jax.experimental.pallas import tpu as pltpu
from jax.experimental.pallas import tpu_sc as plsc

VOCAB = 1_000_000
DIM = 64

NC = 2
NS = 16
NW = NC * NS

CH = 128


def _emb_body(seq, n_chunks, w2, xT, outT, ia_all,
              idx2_0, idx2_1, idx2_2, idx2_3,
              cb_0, cb_1, cb_2, cb_3,
              rows0, rows1, rows2, rows3, tr0, tr1,
              gsem0, gsem1, gsem2, gsem3, osem0, osem1):
    wid = lax.axis_index("s") * NC + lax.axis_index("c")
    cpw = n_chunks // NW
    c_base = wid * cpw
    srows = (seq + 7) // 8 * 8
    iota = lax.iota(jnp.int32, 16)

    for j in range(cpw):
        pltpu.sync_copy(
            xT.at[:, pl.ds((c_base + j) * CH, CH)],
            ia_all.at[pl.ds(j * srows, seq), :],
        )

    def prep(row, idx2b, cbb, rowsb, gsemb):
        for g in range(CH // 16):
            vi = ia_all[row, pl.ds(g * 16, 16)]
            idx2b[pl.ds(g * 16, 16)] = lax.shift_right_logical(vi, 1)
            cbb[pl.ds(g * 16, 16)] = lax.shift_left(
                lax.bitwise_and(vi, 1), 6)
        return pltpu.async_copy(w2.at[idx2b], rowsb, gsemb)

    def transpose(cbb, rowsb, trb):
        @pl.loop(0, CH // 16, unroll=2)
        def _g(g):
            g16 = g * 16
            rid = g16 + iota
            cb = cbb[pl.ds(g16, 16)]
            for d0 in range(0, DIM, 16):
                cbd = cb + d0
                for k in range(16):
                    perm = lax.bitwise_and(iota + k, 15)
                    vals = plsc.load_gather(rowsb, [rid, cbd + perm])
                    plsc.store_scatter(trb, [perm + d0, rid], vals)

    def fire_out(s, cglob, trb, osemb):
        return pltpu.async_copy(
            trb, outT.at[s, :, pl.ds(cglob * CH, CH)], osemb)

    def drain_out(trb, osemb):
        pltpu.make_async_copy(
            trb, outT.at[0, :, pl.ds(0, CH)], osemb).wait()

    def task_coords(t):
        j = t // seq
        s = t - j * seq
        return j, s

    idx2s = [idx2_0, idx2_1, idx2_2, idx2_3]
    cbbs = [cb_0, cb_1, cb_2, cb_3]
    rowss = [rows0, rows1, rows2, rows3]
    trs = [tr0, tr1]
    gsems = [gsem0, gsem1, gsem2, gsem3]
    osems = [osem0, osem1]

    @pl.loop(0, (cpw * seq) // 4)
    def _quad(i):
        staged = []
        for k in range(4):
            t = 4 * i + k
            j, s = task_coords(t)
            d = prep(j * srows + s, idx2s[k], cbbs[k], rowss[k], gsems[k])
            staged.append((s, c_base + j, d))
        for k in range(4):
            s, cglob, d = staged[k]
            d.wait()
            if k < 2:
                @pl.when(i > 0)
                def _():
                    drain_out(trs[k % 2], osems[k % 2])
            else:
                drain_out(trs[k % 2], osems[k % 2])
            transpose(cbbs[k], rowss[k], trs[k % 2])
            fire_out(s, cglob, trs[k % 2], osems[k % 2])

    drain_out(tr0, osem0)
    drain_out(tr1, osem1)


@functools.partial(jax.jit, static_argnames=("seq", "n_chunks"))
def _emb(w2, xT, seq, n_chunks):
    mesh = plsc.VectorSubcoreMesh(
        core_axis_name="c", subcore_axis_name="s", num_cores=NC, num_subcores=NS
    )
    batch = n_chunks * CH
    srows = (seq + 7) // 8 * 8
    cpw = n_chunks // NW
    return pl.kernel(
        functools.partial(_emb_body, seq, n_chunks),
        out_type=jax.ShapeDtypeStruct((seq, DIM, batch), jnp.float32),
        mesh=mesh,
        scratch_types=[
            pltpu.VMEM((cpw * srows, CH), jnp.int32),
            pltpu.VMEM((CH,), jnp.int32),
            pltpu.VMEM((CH,), jnp.int32),
            pltpu.VMEM((CH,), jnp.int32),
            pltpu.VMEM((CH,), jnp.int32),
            pltpu.VMEM((CH,), jnp.int32),
            pltpu.VMEM((CH,), jnp.int32),
            pltpu.VMEM((CH,), jnp.int32),
            pltpu.VMEM((CH,), jnp.int32),
            pltpu.VMEM((CH, 128), jnp.float32),
            pltpu.VMEM((CH, 128), jnp.float32),
            pltpu.VMEM((CH, 128), jnp.float32),
            pltpu.VMEM((CH, 128), jnp.float32),
            pltpu.VMEM((DIM, CH), jnp.float32),
            pltpu.VMEM((DIM, CH), jnp.float32),
            pltpu.SemaphoreType.DMA,
            pltpu.SemaphoreType.DMA,
            pltpu.SemaphoreType.DMA,
            pltpu.SemaphoreType.DMA,
            pltpu.SemaphoreType.DMA,
            pltpu.SemaphoreType.DMA,
        ],
        compiler_params=pltpu.CompilerParams(
            use_tc_tiling_on_sc=True, needs_layout_passes=False),
    )(w2, xT)


def kernel(x, weight):
    b, s = x.shape
    assert b % CH == 0 and (b // CH) % NW == 0 and s % 2 == 0
    w2 = jnp.reshape(weight, (VOCAB // 2, 2 * DIM))
    xT = x.T.astype(jnp.int32)
    outT = _emb(w2, xT, s, b // CH)
    return outT.transpose(2, 0, 1)
